# baseline (speedup 1.0000x reference)
import functools

import jax
import jax.numpy as jnp
from jax import lax
from jax.experimental import pallas as pl
from jax.experimental.pallas import tpu as pltpu
from jax.experimental.pallas import tpu_sc as plsc

D = 64
LB = 128
NB = 4
NC = 2
NS = 16
NW = NC * NS


@functools.lru_cache(maxsize=None)
def _build(batch: int, seq: int):
    assert batch // NW == LB and seq % (2 * NB) == 0
    ngrp2 = seq // (2 * NB)

    mesh = plsc.VectorSubcoreMesh(core_axis_name="c", subcore_axis_name="s")

    @functools.partial(
        pl.kernel,
        mesh=mesh,
        out_type=jax.ShapeDtypeStruct(
            (seq, D // 8, batch // LB, 8, LB), jnp.float32
        ),
        compiler_params=pltpu.CompilerParams(
            use_tc_tiling_on_sc=False, needs_layout_passes=False
        ),
        scratch_types=(
            [
                pltpu.VMEM((seq, LB), jnp.int32),
                pltpu.VMEM((2 * NB, LB, D), jnp.float32),
                pltpu.VMEM((NB, D // 8, 8, LB), jnp.float32),
            ]
            + [pltpu.SemaphoreType.DMA] * (2 * NB)
            + [pltpu.SemaphoreType.DMA] * NB
        ),
    )
    def run(xt_hbm, table_hbm, out_hbm, idx_t, gbuf, tbuf, *sems):
        sem_g = sems[: 2 * NB]
        sem_s = sems[2 * NB :]
        wid = lax.axis_index("s") * NC + lax.axis_index("c")
        col0 = wid * LB
        pltpu.sync_copy(xt_hbm.at[:, pl.ds(col0, LB)], idx_t)

        rowvecs = [
            jnp.arange(16, dtype=jnp.int32) + 16 * c for c in range(8)
        ]

        def transpose_block(gb, tb):
            def trow(i, carry):
                for k in range(8):
                    d = i * 8 + k
                    dvec = jnp.full((16,), 1, jnp.int32) * d
                    for c in range(8):
                        vals = plsc.load_gather(
                            gbuf.at[gb], [rowvecs[c], dvec]
                        )
                        tbuf[tb, i, k, pl.ds(16 * c, 16)] = vals
                return carry

            lax.fori_loop(0, D // 8, trow, 0)

        def start_gather(s, b):
            return pltpu.async_copy(
                table_hbm.at[idx_t.at[s]], gbuf.at[b], sem_g[b]
            )

        def start_scatter(b, s):
            pltpu.async_copy(
                tbuf.at[b], out_hbm.at[s, :, wid], sem_s[b]
            )

        def wait_gather(b):
            pltpu.make_async_copy(
                table_hbm.at[idx_t.at[0]], gbuf.at[b], sem_g[b]
            ).wait()

        def wait_scatter(b):
            pltpu.make_async_copy(
                tbuf.at[b], out_hbm.at[0, :, wid], sem_s[b]
            ).wait()

        for b in range(NB):
            start_gather(b, b)

        def body(G, carry):
            g0 = 2 * G * NB
            for b in range(NB):
                start_gather(g0 + NB + b, NB + b)
            for b in range(NB):
                wait_gather(b)

                @pl.when(G > 0)
                def _():
                    wait_scatter(b)

                transpose_block(b, b)
                start_scatter(b, g0 + b)
            @pl.when(G < ngrp2 - 1)
            def _():
                for b in range(NB):
                    start_gather(g0 + 2 * NB + b, b)

            for b in range(NB):
                wait_gather(NB + b)
                wait_scatter(b)
                transpose_block(NB + b, b)
                start_scatter(b, g0 + NB + b)
            return carry

        lax.fori_loop(0, ngrp2, body, 0)
        for b in range(NB):
            wait_scatter(b)

    return run


def kernel(x, table):
    B, S = x.shape
    V, _ = table.shape
    tp = jnp.pad(table, ((0, 0), (0, 128 - D))).reshape(2 * V, D)
    xt = (x.astype(jnp.int32) * 2).T
    ko = _build(B, S)(xt, tp)
    return ko.transpose(2, 4, 0, 1, 3).reshape(B, S, D)


# speedup vs baseline: 1.7470x; 1.7470x over previous
import functools

import jax
import jax.numpy as jnp
from jax import lax
from jax.experimental import pallas as pl
from jax.experimental.pallas import tpu as pltpu
from jax.experimental.pallas import tpu_sc as plsc

D = 64
W = 128
SPLIT = (0, 96)
SIZES = (96, 104)
NBUF = 4
NC = 2
NS = 16
NW = NC * NS


@functools.lru_cache(maxsize=None)
def _build(batch: int, seq: int):
    rows_per_w = batch // NW
    assert rows_per_w % NBUF == 0 and seq == sum(SIZES)
    ngrp = rows_per_w // NBUF

    mesh = plsc.VectorSubcoreMesh(core_axis_name="c", subcore_axis_name="s")

    @functools.partial(
        pl.kernel,
        mesh=mesh,
        out_type=jax.ShapeDtypeStruct((batch, seq, W), jnp.float32),
        compiler_params=pltpu.CompilerParams(use_tc_tiling_on_sc=False),
        scratch_types=(
            [
                pltpu.VMEM((rows_per_w, seq), jnp.int32),
                pltpu.VMEM((NBUF, seq, W), jnp.float32),
            ]
            + [pltpu.SemaphoreType.DMA] * (2 * NBUF)
        ),
    )
    def run(x_hbm, table_hbm, out_hbm, idx_v, rows_v, *sems):
        sem_g = sems[:NBUF]
        sem_s = sems[NBUF:]
        wid = lax.axis_index("s") * NC + lax.axis_index("c")
        base = wid * rows_per_w
        pltpu.sync_copy(x_hbm.at[pl.ds(base, rows_per_w)], idx_v)

        def group(g, carry):
            for b in range(NBUF):

                @pl.when(g > 0)
                def _():
                    pltpu.make_async_copy(
                        rows_v.at[b], out_hbm.at[0], sem_s[b]
                    ).wait()

            gathers = []
            for b in range(NBUF):
                r = g * NBUF + b
                for off, sz in zip(SPLIT, SIZES):
                    gathers.append(
                        pltpu.async_copy(
                            table_hbm.at[idx_v.at[r, pl.ds(off, sz)]],
                            rows_v.at[b, pl.ds(off, sz)],
                            sem_g[b],
                        )
                    )
            for b in range(NBUF):
                r = g * NBUF + b
                gathers[2 * b].wait()
                gathers[2 * b + 1].wait()
                pltpu.async_copy(rows_v.at[b], out_hbm.at[base + r], sem_s[b])
            return carry

        lax.fori_loop(0, ngrp, group, 0)
        for b in range(NBUF):
            pltpu.make_async_copy(
                rows_v.at[b], out_hbm.at[0], sem_s[b]
            ).wait()

    return run


def kernel(x, table):
    B, S = x.shape
    V, _ = table.shape
    tp = jnp.pad(table, ((0, 0), (0, W - D)))
    ko = _build(B, S)(x.astype(jnp.int32), tp)
    return ko[:, :, :D]
